# baseline (speedup 1.0000x reference)
import functools

import jax
import jax.numpy as jnp
from jax import lax
from jax.experimental import pallas as pl
from jax.experimental.pallas import tpu as pltpu
from jax.experimental.pallas import tpu_sc as plsc

_N = 10000
_E = 160000
_FEAT = 256
_FP = 2048
_RADIUS = 3
_HALF = 128

_NSUB = 16
_EPT = _E // _NSUB
_K = 50
_NCHUNK = _EPT // _K
_PHASES = 5
_PHCH = _NCHUNK // _PHASES
_NBUF = 4
_RPT = 624
_RLAST = _N - 15 * _RPT

_BN = 400
_GRID = _N // _BN


def _sc_agg_body(x0, x1, src2, dst2, out0, out1, srcv, dstv,
                 rows0, rows1, rows2, rows3,
                 acc, g0, g1, g2, g3, s0, s1, s2, s3):
  cid = lax.axis_index("c")
  sid = lax.axis_index("s")
  r0 = pl.multiple_of(sid * _RPT, 8)

  def rowcopy(src_ref, dst_ref):
    @pl.when(sid < 15)
    def _():
      pltpu.sync_copy(src_ref.at[pl.ds(r0, _RPT)], dst_ref.at[pl.ds(r0, _RPT)])

    @pl.when(sid == 15)
    def _():
      pltpu.sync_copy(
          src_ref.at[pl.ds(15 * _RPT, _RLAST)],
          dst_ref.at[pl.ds(15 * _RPT, _RLAST)],
      )

  @pl.when(cid == 0)
  def _():
    rowcopy(x0, acc)

  @pl.when(cid != 0)
  def _():
    rowcopy(x1, acc)

  plsc.subcore_barrier()

  rows = [rows0, rows1, rows2, rows3]
  gs = [g0, g1, g2, g3]
  ss = [s0, s1, s2, s3]

  def run_edges(xh):
    def gat(c, b):
      pltpu.async_copy(xh.at[srcv.at[c]], rows[b], gs[b])

    def wait_g(b):
      pltpu.make_async_copy(xh.at[srcv.at[0]], rows[b], gs[b]).wait()

    def sct(c, b):
      pltpu.async_copy(rows[b], acc.at[dstv.at[c]], ss[b], add=True)

    def wait_s(b):
      pltpu.make_async_copy(rows[b], acc.at[dstv.at[0]], ss[b]).wait()

    for p in range(_PHASES):
      c0 = pl.multiple_of(sid * _NCHUNK + p * _PHCH, 8)
      pltpu.sync_copy(src2.at[pl.ds(c0, _PHCH)], srcv)
      pltpu.sync_copy(dst2.at[pl.ds(c0, _PHCH)], dstv)
      gat(0, 0)
      gat(1, 1)

      def group(t, carry):
        for k in (0, 1):
          c = 4 * t + k
          wait_g(k)
          sct(c, k)

          @pl.when(t >= 1)
          def _():
            wait_s(k + 2)

          gat(c + 2, k + 2)
        for k in (2, 3):
          c = 4 * t + k
          wait_g(k)
          sct(c, k)

          @pl.when(t <= (_PHCH // 4) - 2)
          def _():
            wait_s(k - 2)
            gat(c + 2, k - 2)

        return carry

      lax.fori_loop(0, _PHCH // 4, group, 0)
      for b in range(_NBUF):
        wait_s(b)

  @pl.when(cid == 0)
  def _():
    run_edges(x0)

  @pl.when(cid != 0)
  def _():
    run_edges(x1)

  plsc.subcore_barrier()

  @pl.when(cid == 0)
  def _():
    rowcopy(acc, out0)

  @pl.when(cid != 0)
  def _():
    rowcopy(acc, out1)


@functools.cache
def _get_sc_agg():
  return pl.kernel(
      _sc_agg_body,
      out_type=(
          jax.ShapeDtypeStruct((_N, _HALF), jnp.float32),
          jax.ShapeDtypeStruct((_N, _HALF), jnp.float32),
      ),
      mesh=plsc.VectorSubcoreMesh(core_axis_name="c", subcore_axis_name="s"),
      scratch_types=[
          pltpu.VMEM((_PHCH, _K), jnp.int32),
          pltpu.VMEM((_PHCH, _K), jnp.int32),
          pltpu.VMEM((_K, _HALF), jnp.float32),
          pltpu.VMEM((_K, _HALF), jnp.float32),
          pltpu.VMEM((_K, _HALF), jnp.float32),
          pltpu.VMEM((_K, _HALF), jnp.float32),
          pltpu.VMEM_SHARED((_N, _HALF), jnp.float32),
      ] + [pltpu.SemaphoreType.DMA] * 8,
  )


def _tc_h_body(agg0, agg1, w1t, b1r, h0, h1):
  h = jnp.dot(agg0[...], w1t[:_HALF, :], preferred_element_type=jnp.float32)
  h = h + jnp.dot(agg1[...], w1t[_HALF:, :], preferred_element_type=jnp.float32)
  h = jnp.maximum(h + b1r[...], 0.0)
  h0[...] = h[:, :_HALF]
  h1[...] = h[:, _HALF:]


_tc_h = pl.pallas_call(
    _tc_h_body,
    grid=(_GRID,),
    in_specs=[
        pl.BlockSpec((_BN, _HALF), lambda i: (i, 0)),
        pl.BlockSpec((_BN, _HALF), lambda i: (i, 0)),
        pl.BlockSpec((_FEAT, _FEAT), lambda i: (0, 0)),
        pl.BlockSpec((1, _FEAT), lambda i: (0, 0)),
    ],
    out_specs=[
        pl.BlockSpec((_BN, _HALF), lambda i: (i, 0)),
        pl.BlockSpec((_BN, _HALF), lambda i: (i, 0)),
    ],
    out_shape=[
        jax.ShapeDtypeStruct((_N, _HALF), jnp.float32),
        jax.ShapeDtypeStruct((_N, _HALF), jnp.float32),
    ],
)


def _tc_fp_body(h0, h1, w2t, b2r, fp):
  z = jnp.dot(h0[...], w2t[:_HALF, :], preferred_element_type=jnp.float32)
  z = z + jnp.dot(h1[...], w2t[_HALF:, :], preferred_element_type=jnp.float32)
  z = z + b2r[...]
  m = jnp.max(z, axis=1, keepdims=True)
  e = jnp.exp(z - m)
  s = jnp.sum(e, axis=1, keepdims=True)
  col = jnp.sum(e / s, axis=0, keepdims=True)

  @pl.when(pl.program_id(0) == 0)
  def _():
    fp[...] = jnp.zeros_like(fp)

  fp[...] += col


_tc_fp = pl.pallas_call(
    _tc_fp_body,
    grid=(_GRID,),
    in_specs=[
        pl.BlockSpec((_BN, _HALF), lambda i: (i, 0)),
        pl.BlockSpec((_BN, _HALF), lambda i: (i, 0)),
        pl.BlockSpec((_FEAT, _FP), lambda i: (0, 0)),
        pl.BlockSpec((1, _FP), lambda i: (0, 0)),
    ],
    out_specs=[
        pl.BlockSpec((1, _FP), lambda i: (0, 0)),
    ],
    out_shape=[
        jax.ShapeDtypeStruct((1, _FP), jnp.float32),
    ],
)


@jax.jit
def kernel(x, edge_index, W1, b1, W2, b2):
  src = edge_index[0].reshape(_E // _K, _K)
  dst = edge_index[1].reshape(_E // _K, _K)
  w1t = W1.T
  w2t = W2.T
  b1r = b1.reshape(1, _FEAT)
  b2r = b2.reshape(1, _FP)
  f0 = x[:, :_HALF]
  f1 = x[:, _HALF:]
  fp = jnp.zeros((1, _FP), jnp.float32)
  sc_agg = _get_sc_agg()
  for _ in range(_RADIUS):
    a0, a1 = sc_agg(f0, f1, src, dst)
    f0, f1 = _tc_h(a0, a1, w1t, b1r)
    (fpp,) = _tc_fp(f0, f1, w2t, b2r)
    fp = fp + fpp
  return fp


# speedup vs baseline: 1.2657x; 1.2657x over previous
import functools

import jax
import jax.numpy as jnp
from jax import lax
from jax.experimental import pallas as pl
from jax.experimental.pallas import tpu as pltpu
from jax.experimental.pallas import tpu_sc as plsc

_N = 10000
_E = 160000
_FEAT = 256
_FP = 2048
_RADIUS = 3
_HALF = 128

_NSUB = 16
_EPT = _E // _NSUB
_K = 125
_NCHUNK = _EPT // _K
_PHASES = 2
_PHCH = _NCHUNK // _PHASES
_PHPAIR = _PHCH // 2
_RPT = 624
_RLAST = _N - 15 * _RPT

_BNH = 2000
_GRIDH = _N // _BNH
_BNF = 1000
_GRIDF = _N // _BNF


def _sc_agg_body(x0, x1, src2, dst2, out0, out1, srcv, dstv, rows0, rows1,
                 acc, sem0, sem1):
  cid = lax.axis_index("c")
  sid = lax.axis_index("s")
  r0 = pl.multiple_of(sid * _RPT, 8)

  def rowcopy(src_ref, dst_ref):
    @pl.when(sid < 15)
    def _():
      pltpu.sync_copy(src_ref.at[pl.ds(r0, _RPT)], dst_ref.at[pl.ds(r0, _RPT)])

    @pl.when(sid == 15)
    def _():
      pltpu.sync_copy(
          src_ref.at[pl.ds(15 * _RPT, _RLAST)],
          dst_ref.at[pl.ds(15 * _RPT, _RLAST)],
      )

  @pl.when(cid == 0)
  def _():
    rowcopy(x0, acc)

  @pl.when(cid != 0)
  def _():
    rowcopy(x1, acc)

  plsc.subcore_barrier()

  def run_edges(xh):
    for p in range(_PHASES):
      c0 = pl.multiple_of(sid * _NCHUNK + p * _PHCH, 8)
      pltpu.sync_copy(src2.at[pl.ds(c0, _PHCH)], srcv)
      pltpu.sync_copy(dst2.at[pl.ds(c0, _PHCH)], dstv)
      pltpu.async_copy(xh.at[srcv.at[0]], rows0, sem0)

      def pair(t, c):
        a = 2 * t
        pltpu.async_copy(xh.at[srcv.at[a + 1]], rows1, sem1)
        pltpu.make_async_copy(xh.at[srcv.at[0]], rows0, sem0).wait()
        pltpu.sync_copy(rows0, acc.at[dstv.at[a]], add=True)

        @pl.when(t + 1 < _PHPAIR)
        def _():
          pltpu.async_copy(xh.at[srcv.at[a + 2]], rows0, sem0)

        pltpu.make_async_copy(xh.at[srcv.at[0]], rows1, sem1).wait()
        pltpu.sync_copy(rows1, acc.at[dstv.at[a + 1]], add=True)
        return c

      lax.fori_loop(0, _PHPAIR, pair, 0)

  @pl.when(cid == 0)
  def _():
    run_edges(x0)

  @pl.when(cid != 0)
  def _():
    run_edges(x1)

  plsc.subcore_barrier()

  @pl.when(cid == 0)
  def _():
    rowcopy(acc, out0)

  @pl.when(cid != 0)
  def _():
    rowcopy(acc, out1)


@functools.cache
def _get_sc_agg():
  return pl.kernel(
      _sc_agg_body,
      out_type=(
          jax.ShapeDtypeStruct((_N, _HALF), jnp.float32),
          jax.ShapeDtypeStruct((_N, _HALF), jnp.float32),
      ),
      mesh=plsc.VectorSubcoreMesh(core_axis_name="c", subcore_axis_name="s"),
      scratch_types=[
          pltpu.VMEM((_PHCH, _K), jnp.int32),
          pltpu.VMEM((_PHCH, _K), jnp.int32),
          pltpu.VMEM((_K, _HALF), jnp.float32),
          pltpu.VMEM((_K, _HALF), jnp.float32),
          pltpu.VMEM_SHARED((_N, _HALF), jnp.float32),
          pltpu.SemaphoreType.DMA,
          pltpu.SemaphoreType.DMA,
      ],
  )


def _tc_h_body(agg0, agg1, w1t, b1r, h0, h1):
  h = jnp.dot(agg0[...], w1t[:_HALF, :], preferred_element_type=jnp.float32)
  h = h + jnp.dot(agg1[...], w1t[_HALF:, :], preferred_element_type=jnp.float32)
  h = jnp.maximum(h + b1r[...], 0.0)
  h0[...] = h[:, :_HALF]
  h1[...] = h[:, _HALF:]


_tc_h = pl.pallas_call(
    _tc_h_body,
    grid=(_GRIDH,),
    in_specs=[
        pl.BlockSpec((_BNH, _HALF), lambda i: (i, 0)),
        pl.BlockSpec((_BNH, _HALF), lambda i: (i, 0)),
        pl.BlockSpec((_FEAT, _FEAT), lambda i: (0, 0)),
        pl.BlockSpec((1, _FEAT), lambda i: (0, 0)),
    ],
    out_specs=[
        pl.BlockSpec((_BNH, _HALF), lambda i: (i, 0)),
        pl.BlockSpec((_BNH, _HALF), lambda i: (i, 0)),
    ],
    out_shape=[
        jax.ShapeDtypeStruct((_N, _HALF), jnp.float32),
        jax.ShapeDtypeStruct((_N, _HALF), jnp.float32),
    ],
)


def _tc_fp_body(h0, h1, w2t, b2r, fp):
  z = jnp.dot(h0[...], w2t[:_HALF, :], preferred_element_type=jnp.float32)
  z = z + jnp.dot(h1[...], w2t[_HALF:, :], preferred_element_type=jnp.float32)
  z = z + b2r[...]
  m = jnp.max(z, axis=1, keepdims=True)
  e = jnp.exp(z - m)
  s = jnp.sum(e, axis=1, keepdims=True)
  col = jnp.sum(e / s, axis=0, keepdims=True)

  @pl.when(pl.program_id(0) == 0)
  def _():
    fp[...] = jnp.zeros_like(fp)

  fp[...] += col


_tc_fp = pl.pallas_call(
    _tc_fp_body,
    grid=(_GRIDF,),
    in_specs=[
        pl.BlockSpec((_BNF, _HALF), lambda i: (i, 0)),
        pl.BlockSpec((_BNF, _HALF), lambda i: (i, 0)),
        pl.BlockSpec((_FEAT, _FP), lambda i: (0, 0)),
        pl.BlockSpec((1, _FP), lambda i: (0, 0)),
    ],
    out_specs=[
        pl.BlockSpec((1, _FP), lambda i: (0, 0)),
    ],
    out_shape=[
        jax.ShapeDtypeStruct((1, _FP), jnp.float32),
    ],
)


@jax.jit
def kernel(x, edge_index, W1, b1, W2, b2):
  src = edge_index[0].reshape(_E // _K, _K)
  dst = edge_index[1].reshape(_E // _K, _K)
  w1t = W1.T
  w2t = W2.T
  b1r = b1.reshape(1, _FEAT)
  b2r = b2.reshape(1, _FP)
  f0 = x[:, :_HALF]
  f1 = x[:, _HALF:]
  fp = jnp.zeros((1, _FP), jnp.float32)
  sc_agg = _get_sc_agg()
  for _ in range(_RADIUS):
    a0, a1 = sc_agg(f0, f1, src, dst)
    f0, f1 = _tc_h(a0, a1, w1t, b1r)
    (fpp,) = _tc_fp(f0, f1, w2t, b2r)
    fp = fp + fpp
  return fp


# speedup vs baseline: 1.2748x; 1.0072x over previous
import functools

import jax
import jax.numpy as jnp
from jax import lax
from jax.experimental import pallas as pl
from jax.experimental.pallas import tpu as pltpu
from jax.experimental.pallas import tpu_sc as plsc

_N = 10000
_E = 160000
_FEAT = 256
_FP = 2048
_RADIUS = 3
_HALF = 128

_NSUB = 16
_EPT = _E // _NSUB
_K = 125
_NCHUNK = _EPT // _K
_PHASES = 2
_PHCH = _NCHUNK // _PHASES
_PHPAIR = _PHCH // 2
_RPT = 624
_RLAST = _N - 15 * _RPT

_BNH = 2000
_GRIDH = _N // _BNH
_BNF = 1000
_GRIDF = _N // _BNF


def _sc_agg_body(x0, x1, src2, dst2, out0, out1, srcv, dstv, rows0, rows1,
                 acc, sem0, sem1, sem2):
  cid = lax.axis_index("c")
  sid = lax.axis_index("s")
  r0 = pl.multiple_of(sid * _RPT, 8)

  def rowcopy(src_ref, dst_ref):
    @pl.when(sid < 15)
    def _():
      pltpu.sync_copy(src_ref.at[pl.ds(r0, _RPT)], dst_ref.at[pl.ds(r0, _RPT)])

    @pl.when(sid == 15)
    def _():
      pltpu.sync_copy(
          src_ref.at[pl.ds(15 * _RPT, _RLAST)],
          dst_ref.at[pl.ds(15 * _RPT, _RLAST)],
      )

  def rowcopy_async(src_ref, dst_ref, wait):
    @pl.when(sid < 15)
    def _():
      d = pltpu.make_async_copy(
          src_ref.at[pl.ds(r0, _RPT)], dst_ref.at[pl.ds(r0, _RPT)], sem2)
      d.wait() if wait else d.start()

    @pl.when(sid == 15)
    def _():
      d = pltpu.make_async_copy(
          src_ref.at[pl.ds(15 * _RPT, _RLAST)],
          dst_ref.at[pl.ds(15 * _RPT, _RLAST)], sem2)
      d.wait() if wait else d.start()

  c00 = pl.multiple_of(sid * _NCHUNK, 8)
  pltpu.sync_copy(src2.at[pl.ds(c00, _PHCH)], srcv)
  pltpu.sync_copy(dst2.at[pl.ds(c00, _PHCH)], dstv)

  @pl.when(cid == 0)
  def _():
    rowcopy_async(x0, acc, False)
    pltpu.async_copy(x0.at[srcv.at[0]], rows0, sem0)

  @pl.when(cid != 0)
  def _():
    rowcopy_async(x1, acc, False)
    pltpu.async_copy(x1.at[srcv.at[0]], rows0, sem0)

  @pl.when(cid == 0)
  def _():
    rowcopy_async(x0, acc, True)

  @pl.when(cid != 0)
  def _():
    rowcopy_async(x1, acc, True)

  plsc.subcore_barrier()

  def run_edges(xh):
    for p in range(_PHASES):
      if p > 0:
        c0 = pl.multiple_of(sid * _NCHUNK + p * _PHCH, 8)
        pltpu.sync_copy(src2.at[pl.ds(c0, _PHCH)], srcv)
        pltpu.sync_copy(dst2.at[pl.ds(c0, _PHCH)], dstv)
        pltpu.async_copy(xh.at[srcv.at[0]], rows0, sem0)

      def pair(t, c):
        a = 2 * t
        pltpu.async_copy(xh.at[srcv.at[a + 1]], rows1, sem1)
        pltpu.make_async_copy(xh.at[srcv.at[0]], rows0, sem0).wait()
        pltpu.sync_copy(rows0, acc.at[dstv.at[a]], add=True)

        @pl.when(t + 1 < _PHPAIR)
        def _():
          pltpu.async_copy(xh.at[srcv.at[a + 2]], rows0, sem0)

        pltpu.make_async_copy(xh.at[srcv.at[0]], rows1, sem1).wait()
        pltpu.sync_copy(rows1, acc.at[dstv.at[a + 1]], add=True)
        return c

      lax.fori_loop(0, _PHPAIR, pair, 0)

  @pl.when(cid == 0)
  def _():
    run_edges(x0)

  @pl.when(cid != 0)
  def _():
    run_edges(x1)

  plsc.subcore_barrier()

  @pl.when(cid == 0)
  def _():
    rowcopy(acc, out0)

  @pl.when(cid != 0)
  def _():
    rowcopy(acc, out1)


@functools.cache
def _get_sc_agg():
  return pl.kernel(
      _sc_agg_body,
      out_type=(
          jax.ShapeDtypeStruct((_N, _HALF), jnp.float32),
          jax.ShapeDtypeStruct((_N, _HALF), jnp.float32),
      ),
      mesh=plsc.VectorSubcoreMesh(core_axis_name="c", subcore_axis_name="s"),
      scratch_types=[
          pltpu.VMEM((_PHCH, _K), jnp.int32),
          pltpu.VMEM((_PHCH, _K), jnp.int32),
          pltpu.VMEM((_K, _HALF), jnp.float32),
          pltpu.VMEM((_K, _HALF), jnp.float32),
          pltpu.VMEM_SHARED((_N, _HALF), jnp.float32),
          pltpu.SemaphoreType.DMA,
          pltpu.SemaphoreType.DMA,
          pltpu.SemaphoreType.DMA,
      ],
  )


def _tc_h_body(agg0, agg1, w1t, b1r, h0, h1):
  h = jnp.dot(agg0[...], w1t[:_HALF, :], preferred_element_type=jnp.float32)
  h = h + jnp.dot(agg1[...], w1t[_HALF:, :], preferred_element_type=jnp.float32)
  h = jnp.maximum(h + b1r[...], 0.0)
  h0[...] = h[:, :_HALF]
  h1[...] = h[:, _HALF:]


_tc_h = pl.pallas_call(
    _tc_h_body,
    grid=(_GRIDH,),
    in_specs=[
        pl.BlockSpec((_BNH, _HALF), lambda i: (i, 0)),
        pl.BlockSpec((_BNH, _HALF), lambda i: (i, 0)),
        pl.BlockSpec((_FEAT, _FEAT), lambda i: (0, 0)),
        pl.BlockSpec((1, _FEAT), lambda i: (0, 0)),
    ],
    out_specs=[
        pl.BlockSpec((_BNH, _HALF), lambda i: (i, 0)),
        pl.BlockSpec((_BNH, _HALF), lambda i: (i, 0)),
    ],
    out_shape=[
        jax.ShapeDtypeStruct((_N, _HALF), jnp.float32),
        jax.ShapeDtypeStruct((_N, _HALF), jnp.float32),
    ],
)


def _tc_fp_body(h0, h1, w2t, b2r, fp):
  hb0 = h0[...].astype(jnp.bfloat16)
  hb1 = h1[...].astype(jnp.bfloat16)
  z = jnp.dot(hb0, w2t[:_HALF, :], preferred_element_type=jnp.float32)
  z = z + jnp.dot(hb1, w2t[_HALF:, :], preferred_element_type=jnp.float32)
  z = z + b2r[...]
  m = jnp.max(z, axis=1, keepdims=True)
  e = jnp.exp(z - m)
  s = jnp.sum(e, axis=1, keepdims=True)
  col = jnp.sum(e / s, axis=0, keepdims=True)

  @pl.when(pl.program_id(0) == 0)
  def _():
    fp[...] = jnp.zeros_like(fp)

  fp[...] += col


_tc_fp = pl.pallas_call(
    _tc_fp_body,
    grid=(_GRIDF,),
    in_specs=[
        pl.BlockSpec((_BNF, _HALF), lambda i: (i, 0)),
        pl.BlockSpec((_BNF, _HALF), lambda i: (i, 0)),
        pl.BlockSpec((_FEAT, _FP), lambda i: (0, 0)),
        pl.BlockSpec((1, _FP), lambda i: (0, 0)),
    ],
    out_specs=[
        pl.BlockSpec((1, _FP), lambda i: (0, 0)),
    ],
    out_shape=[
        jax.ShapeDtypeStruct((1, _FP), jnp.float32),
    ],
)


@jax.jit
def kernel(x, edge_index, W1, b1, W2, b2):
  src = edge_index[0].reshape(_E // _K, _K)
  dst = edge_index[1].reshape(_E // _K, _K)
  w1t = W1.T
  w2t = W2.T.astype(jnp.bfloat16)
  b1r = b1.reshape(1, _FEAT)
  b2r = b2.reshape(1, _FP)
  f0 = x[:, :_HALF]
  f1 = x[:, _HALF:]
  fp = jnp.zeros((1, _FP), jnp.float32)
  sc_agg = _get_sc_agg()
  for _ in range(_RADIUS):
    a0, a1 = sc_agg(f0, f1, src, dst)
    f0, f1 = _tc_h(a0, a1, w1t, b1r)
    (fpp,) = _tc_fp(f0, f1, w2t, b2r)
    fp = fp + fpp
  return fp


# speedup vs baseline: 1.3272x; 1.0411x over previous
import functools

import jax
import jax.numpy as jnp
from jax import lax
from jax.experimental import pallas as pl
from jax.experimental.pallas import tpu as pltpu
from jax.experimental.pallas import tpu_sc as plsc

_N = 10000
_E = 160000
_FEAT = 256
_FP = 2048
_RADIUS = 3
_HALF = 128

_NSUB = 16
_EPT = _E // _NSUB
_K = 125
_NCHUNK = _EPT // _K
_PHASES = 2
_PHCH = _NCHUNK // _PHASES
_PHPAIR = _PHCH // 2
_RPT = 624
_RLAST = _N - 15 * _RPT

_BNH = 2000
_GRIDH = _N // _BNH
_BNF = 1000
_GRIDF = _N // _BNF


def _sc_agg_body(x0, x1, src2, dst2, out0, out1, srcv, dstv, rows0, rows1,
                 acc, sem0, sem1, sem2):
  cid = lax.axis_index("c")
  sid = lax.axis_index("s")
  r0 = pl.multiple_of(sid * _RPT, 8)

  def rowcopy(src_ref, dst_ref):
    @pl.when(sid < 15)
    def _():
      pltpu.sync_copy(src_ref.at[pl.ds(r0, _RPT)], dst_ref.at[pl.ds(r0, _RPT)])

    @pl.when(sid == 15)
    def _():
      pltpu.sync_copy(
          src_ref.at[pl.ds(15 * _RPT, _RLAST)],
          dst_ref.at[pl.ds(15 * _RPT, _RLAST)],
      )

  def rowcopy_async(src_ref, dst_ref, wait):
    @pl.when(sid < 15)
    def _():
      d = pltpu.make_async_copy(
          src_ref.at[pl.ds(r0, _RPT)], dst_ref.at[pl.ds(r0, _RPT)], sem2)
      d.wait() if wait else d.start()

    @pl.when(sid == 15)
    def _():
      d = pltpu.make_async_copy(
          src_ref.at[pl.ds(15 * _RPT, _RLAST)],
          dst_ref.at[pl.ds(15 * _RPT, _RLAST)], sem2)
      d.wait() if wait else d.start()

  c00 = pl.multiple_of(sid * _NCHUNK, 8)
  pltpu.sync_copy(src2.at[pl.ds(c00, _PHCH)], srcv)
  pltpu.sync_copy(dst2.at[pl.ds(c00, _PHCH)], dstv)

  @pl.when(cid == 0)
  def _():
    rowcopy_async(x0, acc, False)
    pltpu.async_copy(x0.at[srcv.at[0]], rows0, sem0)

  @pl.when(cid != 0)
  def _():
    rowcopy_async(x1, acc, False)
    pltpu.async_copy(x1.at[srcv.at[0]], rows0, sem0)

  @pl.when(cid == 0)
  def _():
    rowcopy_async(x0, acc, True)

  @pl.when(cid != 0)
  def _():
    rowcopy_async(x1, acc, True)

  plsc.subcore_barrier()

  def run_edges(xh):
    for p in range(_PHASES):
      if p > 0:
        c0 = pl.multiple_of(sid * _NCHUNK + p * _PHCH, 8)
        pltpu.sync_copy(src2.at[pl.ds(c0, _PHCH)], srcv)
        pltpu.sync_copy(dst2.at[pl.ds(c0, _PHCH)], dstv)
        pltpu.async_copy(xh.at[srcv.at[0]], rows0, sem0)

      def pair(t, c):
        a = 2 * t
        pltpu.async_copy(xh.at[srcv.at[a + 1]], rows1, sem1)
        pltpu.make_async_copy(xh.at[srcv.at[0]], rows0, sem0).wait()
        pltpu.sync_copy(rows0, acc.at[dstv.at[a]], add=True)

        @pl.when(t + 1 < _PHPAIR)
        def _():
          pltpu.async_copy(xh.at[srcv.at[a + 2]], rows0, sem0)

        pltpu.make_async_copy(xh.at[srcv.at[0]], rows1, sem1).wait()
        pltpu.sync_copy(rows1, acc.at[dstv.at[a + 1]], add=True)
        return c

      lax.fori_loop(0, _PHPAIR, pair, 0)

  @pl.when(cid == 0)
  def _():
    run_edges(x0)

  @pl.when(cid != 0)
  def _():
    run_edges(x1)

  plsc.subcore_barrier()

  @pl.when(cid == 0)
  def _():
    rowcopy(acc, out0)

  @pl.when(cid != 0)
  def _():
    rowcopy(acc, out1)


@functools.cache
def _get_sc_agg():
  return pl.kernel(
      _sc_agg_body,
      out_type=(
          jax.ShapeDtypeStruct((_N, _HALF), jnp.float32),
          jax.ShapeDtypeStruct((_N, _HALF), jnp.float32),
      ),
      mesh=plsc.VectorSubcoreMesh(core_axis_name="c", subcore_axis_name="s"),
      scratch_types=[
          pltpu.VMEM((_PHCH, _K), jnp.int32),
          pltpu.VMEM((_PHCH, _K), jnp.int32),
          pltpu.VMEM((_K, _HALF), jnp.float32),
          pltpu.VMEM((_K, _HALF), jnp.float32),
          pltpu.VMEM_SHARED((_N, _HALF), jnp.float32),
          pltpu.SemaphoreType.DMA,
          pltpu.SemaphoreType.DMA,
          pltpu.SemaphoreType.DMA,
      ],
  )


def _tc_h_body(agg0, agg1, w1t, b1r, h0, h1):
  h = jnp.dot(agg0[...], w1t[:_HALF, :], preferred_element_type=jnp.float32)
  h = h + jnp.dot(agg1[...], w1t[_HALF:, :], preferred_element_type=jnp.float32)
  h = jnp.maximum(h + b1r[...], 0.0)
  h0[...] = h[:, :_HALF]
  h1[...] = h[:, _HALF:]


_tc_h = pl.pallas_call(
    _tc_h_body,
    grid=(_GRIDH,),
    in_specs=[
        pl.BlockSpec((_BNH, _HALF), lambda i: (i, 0)),
        pl.BlockSpec((_BNH, _HALF), lambda i: (i, 0)),
        pl.BlockSpec((_FEAT, _FEAT), lambda i: (0, 0)),
        pl.BlockSpec((1, _FEAT), lambda i: (0, 0)),
    ],
    out_specs=[
        pl.BlockSpec((_BNH, _HALF), lambda i: (i, 0)),
        pl.BlockSpec((_BNH, _HALF), lambda i: (i, 0)),
    ],
    out_shape=[
        jax.ShapeDtypeStruct((_N, _HALF), jnp.float32),
        jax.ShapeDtypeStruct((_N, _HALF), jnp.float32),
    ],
)


def _tc_last_body(agg0, agg1, w1t, b1r, w2t, b2r, fp):
  h = jnp.dot(agg0[...], w1t[:_HALF, :], preferred_element_type=jnp.float32)
  h = h + jnp.dot(agg1[...], w1t[_HALF:, :], preferred_element_type=jnp.float32)
  h = jnp.maximum(h + b1r[...], 0.0)
  z = jnp.dot(h.astype(jnp.bfloat16), w2t[...],
              preferred_element_type=jnp.float32) + b2r[...]
  m = jnp.max(z, axis=1, keepdims=True)
  e = jnp.exp(z - m)
  s = jnp.sum(e, axis=1, keepdims=True)
  col = jnp.sum(e / s, axis=0, keepdims=True)

  @pl.when(pl.program_id(0) == 0)
  def _():
    fp[...] = jnp.zeros_like(fp)

  fp[...] += col


_tc_last = pl.pallas_call(
    _tc_last_body,
    grid=(_GRIDF,),
    in_specs=[
        pl.BlockSpec((_BNF, _HALF), lambda i: (i, 0)),
        pl.BlockSpec((_BNF, _HALF), lambda i: (i, 0)),
        pl.BlockSpec((_FEAT, _FEAT), lambda i: (0, 0)),
        pl.BlockSpec((1, _FEAT), lambda i: (0, 0)),
        pl.BlockSpec((_FEAT, _FP), lambda i: (0, 0)),
        pl.BlockSpec((1, _FP), lambda i: (0, 0)),
    ],
    out_specs=[
        pl.BlockSpec((1, _FP), lambda i: (0, 0)),
    ],
    out_shape=[
        jax.ShapeDtypeStruct((1, _FP), jnp.float32),
    ],
)


def _tc_fp_body(h0, h1, w2t, b2r, fp):
  hb0 = h0[...].astype(jnp.bfloat16)
  hb1 = h1[...].astype(jnp.bfloat16)
  z = jnp.dot(hb0, w2t[:_HALF, :], preferred_element_type=jnp.float32)
  z = z + jnp.dot(hb1, w2t[_HALF:, :], preferred_element_type=jnp.float32)
  z = z + b2r[...]
  m = jnp.max(z, axis=1, keepdims=True)
  e = jnp.exp(z - m)
  s = jnp.sum(e, axis=1, keepdims=True)
  col = jnp.sum(e / s, axis=0, keepdims=True)

  @pl.when(pl.program_id(0) == 0)
  def _():
    fp[...] = jnp.zeros_like(fp)

  fp[...] += col


_tc_fp = pl.pallas_call(
    _tc_fp_body,
    grid=(_GRIDF,),
    in_specs=[
        pl.BlockSpec((_BNF, _HALF), lambda i: (i, 0)),
        pl.BlockSpec((_BNF, _HALF), lambda i: (i, 0)),
        pl.BlockSpec((_FEAT, _FP), lambda i: (0, 0)),
        pl.BlockSpec((1, _FP), lambda i: (0, 0)),
    ],
    out_specs=[
        pl.BlockSpec((1, _FP), lambda i: (0, 0)),
    ],
    out_shape=[
        jax.ShapeDtypeStruct((1, _FP), jnp.float32),
    ],
)


@jax.jit
def kernel(x, edge_index, W1, b1, W2, b2):
  src = edge_index[0].reshape(_E // _K, _K)
  dst = edge_index[1].reshape(_E // _K, _K)
  w1t = W1.T
  w2t = W2.T.astype(jnp.bfloat16)
  b1r = b1.reshape(1, _FEAT)
  b2r = b2.reshape(1, _FP)
  f0 = x[:, :_HALF]
  f1 = x[:, _HALF:]
  fp = jnp.zeros((1, _FP), jnp.float32)
  sc_agg = _get_sc_agg()
  for r in range(_RADIUS):
    a0, a1 = sc_agg(f0, f1, src, dst)
    if r < _RADIUS - 1:
      f0, f1 = _tc_h(a0, a1, w1t, b1r)
      (fpp,) = _tc_fp(f0, f1, w2t, b2r)
    else:
      (fpp,) = _tc_last(a0, a1, w1t, b1r, w2t, b2r)
    fp = fp + fpp
  return fp
